# baseline (speedup 1.0000x reference)
import jax
import jax.numpy as jnp
import numpy as np
from jax.experimental import pallas as pl
from jax.experimental.pallas import tpu as pltpu

D = 768
H = 12
HKV = 3
HPG = H // HKV
HD = 64
L = 32
STRIDE = 16
TOPN = 16
WIN = 512
S = 2048
NCMP = (S - L) // STRIDE + 1
NCMP_PAD = 128
NBLK = S // L
SCALE = 1.0 / np.sqrt(HD)

F32 = jnp.float32


def _ln(xb, g, b):
    m = jnp.mean(xb, axis=-1, keepdims=True)
    v = jnp.var(xb, axis=-1, keepdims=True)
    return (xb - m) * jax.lax.rsqrt(v + 1e-5) * g + b


def _k1_body(x_ref, g_ref, b_ref, w_ref, bc_ref, q_ref, kv_ref, gt_ref):
    xb = x_ref[:]
    ln = _ln(xb, g_ref[:], b_ref[:])
    out = jnp.dot(ln, w_ref[:], preferred_element_type=F32) + bc_ref[:]
    q_ref[:] = out[:, :D]
    for g in range(HKV):
        kv_ref[g] = jnp.concatenate(
            [out[:, D + g * HD:D + (g + 1) * HD],
             out[:, D + HKV * HD + g * HD:D + HKV * HD + (g + 1) * HD]],
            axis=1)
    gt_ref[:] = jax.nn.sigmoid(out[:, D + 2 * HKV * HD:])


def _k1(x, ln1_g, ln1_b, Wcat, bcat):
    blk = 256
    return pl.pallas_call(
        _k1_body,
        grid=(S // blk,),
        compiler_params=pltpu.CompilerParams(dimension_semantics=("parallel",)),
        in_specs=[
            pl.BlockSpec((blk, D), lambda i: (i, 0)),
            pl.BlockSpec((1, D), lambda i: (0, 0)),
            pl.BlockSpec((1, D), lambda i: (0, 0)),
            pl.BlockSpec(Wcat.shape, lambda i: (0, 0)),
            pl.BlockSpec((1, Wcat.shape[1]), lambda i: (0, 0)),
        ],
        out_specs=[
            pl.BlockSpec((blk, D), lambda i: (i, 0)),
            pl.BlockSpec((HKV, blk, 2 * HD), lambda i: (0, i, 0)),
            pl.BlockSpec((blk, 128), lambda i: (i, 0)),
        ],
        out_shape=[
            jax.ShapeDtypeStruct((S, D), F32),
            jax.ShapeDtypeStruct((HKV, S, 2 * HD), F32),
            jax.ShapeDtypeStruct((S, 128), F32),
        ],
    )(x, ln1_g, ln1_b, Wcat, bcat)


QC3 = 512


def _kb_body(q_ref, kvf_ref, w_ref, b_ref, out_ref, idx_ref, ckv_s, impq_s):
    i = pl.program_id(1)
    nsteps = pl.num_programs(1)

    @pl.when(i == 0)
    def _():
        r = kvf_ref[0]
        t0 = jnp.dot(r, w_ref[:STRIDE * 2 * HD], preferred_element_type=F32)
        t1 = jnp.dot(r, w_ref[STRIDE * 2 * HD:], preferred_element_type=F32)
        zero = jnp.zeros((1, 2 * HD), F32)
        ckv_s[:] = t0 + jnp.concatenate([t1[1:], zero], axis=0) + b_ref[:]

    ckm = ckv_s[:, :HD]
    cvm = ckv_s[:, HD:]
    rows = jax.lax.broadcasted_iota(jnp.int32, (HPG * QC3, 1), 0)
    qpos = i * QC3 + rows % QC3
    nidx = jax.lax.broadcasted_iota(jnp.int32, (1, NCMP_PAD), 1)
    mask = qpos >= nidx * STRIDE + (L - 1)
    pad = nidx < NCMP

    q4 = jnp.concatenate(
        [q_ref[:, hp * HD:(hp + 1) * HD] for hp in range(HPG)], axis=0)
    s = jax.lax.dot_general(q4, ckm, (((1,), (1,)), ((), ())),
                            preferred_element_type=F32) * SCALE
    s = jnp.where(mask, s, -1e9)
    s = jnp.where(pad, s, -1e30)
    m = jnp.max(s, axis=-1, keepdims=True)
    p = jnp.exp(s - m)
    cp = p / jnp.sum(p, axis=-1, keepdims=True)
    o = jnp.dot(cp, cvm, preferred_element_type=F32)
    for hp in range(HPG):
        out_ref[:, hp * HD:(hp + 1) * HD] = o[hp * QC3:(hp + 1) * QC3, :]

    nq = QC3 // L
    ar = jax.lax.broadcasted_iota(jnp.int32, (nq, HPG * QC3), 0)
    ac = jax.lax.broadcasted_iota(jnp.int32, (nq, HPG * QC3), 1)
    A = jnp.where((ac % QC3) // L == ar, 1.0, 0.0).astype(F32)
    rr = jax.lax.broadcasted_iota(jnp.int32, (NCMP_PAD, NBLK), 0)
    cc = jax.lax.broadcasted_iota(jnp.int32, (NCMP_PAD, NBLK), 1)
    P = jnp.where((rr // 2 == cc) & (rr < NCMP), 1.0, 0.0).astype(F32)
    folded = jnp.dot(cp, P, preferred_element_type=F32)
    impq_s[pl.ds(i * nq, nq), :] = jnp.dot(A, folded,
                                           preferred_element_type=F32)

    @pl.when(i == nsteps - 1)
    def _():
        vals = impq_s[:]
        qb = jax.lax.broadcasted_iota(jnp.int32, (NBLK, NBLK), 0)
        mb = jax.lax.broadcasted_iota(jnp.int32, (NBLK, NBLK), 1)
        bonus = jnp.where((mb == qb) | (mb == 0), 1e6, 0.0).astype(F32)
        vals = jnp.where(qb >= mb, vals + bonus, -1e9)
        tcol = jax.lax.broadcasted_iota(jnp.int32, (NBLK, TOPN), 1)
        out = jnp.zeros((NBLK, TOPN), jnp.int32)
        for t in range(TOPN):
            m = jnp.argmax(vals, axis=1).astype(jnp.int32)
            out = jnp.where(tcol == t, m[:, None], out)
            vals = jnp.where(mb == m[:, None], -3e9, vals)
        idx_ref[0] = out


def _kb(q, kvf, W_all, bckv):
    return pl.pallas_call(
        _kb_body,
        grid=(HKV, S // QC3),
        compiler_params=pltpu.CompilerParams(
            dimension_semantics=("arbitrary", "arbitrary")),
        in_specs=[
            pl.BlockSpec((QC3, HPG * HD), lambda g, i: (i, g)),
            pl.BlockSpec((1, S // STRIDE, STRIDE * 2 * HD),
                         lambda g, i: (g, 0, 0)),
            pl.BlockSpec(W_all.shape, lambda g, i: (0, 0)),
            pl.BlockSpec((1, 2 * HD), lambda g, i: (0, 0)),
        ],
        out_specs=[
            pl.BlockSpec((QC3, HPG * HD), lambda g, i: (i, g)),
            pl.BlockSpec((1, NBLK, TOPN), lambda g, i: (g, 0, 0)),
        ],
        out_shape=[
            jax.ShapeDtypeStruct((S, D), F32),
            jax.ShapeDtypeStruct((HKV, NBLK, TOPN), jnp.int32),
        ],
        scratch_shapes=[
            pltpu.VMEM((NCMP_PAD, 2 * HD), F32),
            pltpu.VMEM((NBLK, NBLK), F32),
        ],
    )(q, kvf, W_all, bckv)


QC = 256
QB5 = QC // L
WK = WIN + QC


def _kc_body(idx_ref, q_ref, kv_ref, sel_ref, win_ref, kv_scr):
    g = pl.program_id(0)
    i = pl.program_id(1)

    rows = jax.lax.broadcasted_iota(jnp.int32, (HPG * L, 1), 0)
    jcol = jax.lax.broadcasted_iota(jnp.int32, (1, TOPN * L), 1)
    jmod = jcol % L

    for qq in range(QB5):
        qb = i * QB5 + qq
        base = g * NBLK * TOPN + qb * TOPN
        qpos = qb * L + rows % L

        colpos = jmod
        for t in range(TOPN):
            it = idx_ref[base + t]
            kv_scr[qq * TOPN * L + t * L:qq * TOPN * L + (t + 1) * L, :] = (
                kv_ref[0, pl.ds(it * L, L), :])
            colpos = colpos + jnp.where(jcol // L == t, it * L, 0)
        mask = colpos <= qpos

        ks = kv_scr[qq * TOPN * L:(qq + 1) * TOPN * L, :HD]
        vs = kv_scr[qq * TOPN * L:(qq + 1) * TOPN * L, HD:]
        q4 = jnp.concatenate(
            [q_ref[qq * L:(qq + 1) * L, hp * HD:(hp + 1) * HD]
             for hp in range(HPG)], axis=0)
        s = jax.lax.dot_general(q4, ks, (((1,), (1,)), ((), ())),
                                preferred_element_type=F32) * SCALE
        s = jnp.where(mask, s, -1e9)
        m = jnp.max(s, axis=-1, keepdims=True)
        p = jnp.exp(s - m)
        r = 1.0 / jnp.sum(p, axis=-1, keepdims=True)
        o = jnp.dot(p, vs, preferred_element_type=F32) * r
        for hp in range(HPG):
            sel_ref[qq * L:(qq + 1) * L, hp * HD:(hp + 1) * HD] = (
                o[hp * L:(hp + 1) * L, :])

    s0 = jnp.maximum(i - WIN // QC, 0) * QC
    kvw = kv_ref[0, pl.ds(s0, WK), :]
    kw = kvw[:, :HD]
    vw = kvw[:, HD:]
    wrows = jax.lax.broadcasted_iota(jnp.int32, (HPG * QC, 1), 0)
    qpos = i * QC + wrows % QC
    kpos = s0 + jax.lax.broadcasted_iota(jnp.int32, (1, WK), 1)
    wmask = (qpos >= kpos) & (qpos - kpos < WIN)
    q4 = jnp.concatenate(
        [q_ref[:, hp * HD:(hp + 1) * HD] for hp in range(HPG)], axis=0)
    s = jax.lax.dot_general(q4, kw, (((1,), (1,)), ((), ())),
                            preferred_element_type=F32) * SCALE
    s = jnp.where(wmask, s, -1e9)
    m = jnp.max(s, axis=-1, keepdims=True)
    p = jnp.exp(s - m)
    r = 1.0 / jnp.sum(p, axis=-1, keepdims=True)
    o = jnp.dot(p, vw, preferred_element_type=F32) * r
    for hp in range(HPG):
        win_ref[:, hp * HD:(hp + 1) * HD] = o[hp * QC:(hp + 1) * QC, :]


def _kc(top_idx_flat, q, kvh):
    grid_spec = pltpu.PrefetchScalarGridSpec(
        num_scalar_prefetch=1,
        grid=(HKV, S // QC),
        in_specs=[
            pl.BlockSpec((QC, HPG * HD), lambda g, i, *_: (i, g)),
            pl.BlockSpec((1, S, 2 * HD), lambda g, i, *_: (g, 0, 0)),
        ],
        out_specs=[
            pl.BlockSpec((QC, HPG * HD), lambda g, i, *_: (i, g)),
            pl.BlockSpec((QC, HPG * HD), lambda g, i, *_: (i, g)),
        ],
        scratch_shapes=[
            pltpu.VMEM((QB5 * TOPN * L, 2 * HD), F32),
        ],
    )
    return pl.pallas_call(
        _kc_body,
        grid_spec=grid_spec,
        compiler_params=pltpu.CompilerParams(
            dimension_semantics=("parallel", "parallel")),
        out_shape=[
            jax.ShapeDtypeStruct((S, D), F32),
            jax.ShapeDtypeStruct((S, D), F32),
        ],
    )(top_idx_flat, q, kvh)


def _k7_body(x_ref, cmp_ref, sel_ref, win_ref, g_ref, wo_ref,
             ln2g_ref, ln2b_ref, w1_ref, b1_ref, w2_ref, b2_ref, out_ref):
    gts = g_ref[:]
    rr = jax.lax.broadcasted_iota(jnp.int32, (128, D), 0)
    cc = jax.lax.broadcasted_iota(jnp.int32, (128, D), 1)
    head3 = 3 * (cc // HD)
    e0 = jnp.where(rr == head3, 1.0, 0.0).astype(F32)
    e1 = jnp.where(rr == head3 + 1, 1.0, 0.0).astype(F32)
    e2 = jnp.where(rr == head3 + 2, 1.0, 0.0).astype(F32)
    comb = (cmp_ref[:] * jnp.dot(gts, e0, preferred_element_type=F32)
            + sel_ref[:] * jnp.dot(gts, e1, preferred_element_type=F32)
            + win_ref[:] * jnp.dot(gts, e2, preferred_element_type=F32))
    x1 = x_ref[:] + jnp.dot(comb, wo_ref[:], preferred_element_type=F32)
    ln = _ln(x1, ln2g_ref[:], ln2b_ref[:])
    h = jax.nn.gelu(jnp.dot(ln, w1_ref[:], preferred_element_type=F32) + b1_ref[:])
    out_ref[:] = x1 + jnp.dot(h, w2_ref[:], preferred_element_type=F32) + b2_ref[:]


def _k7(x, out_cmp, out_sel, out_win, gates, Wo, ln2_g, ln2_b, W1, b1, W2, b2):
    blk = 256
    return pl.pallas_call(
        _k7_body,
        grid=(S // blk,),
        compiler_params=pltpu.CompilerParams(dimension_semantics=("parallel",)),
        in_specs=[
            pl.BlockSpec((blk, D), lambda i: (i, 0)),
            pl.BlockSpec((blk, D), lambda i: (i, 0)),
            pl.BlockSpec((blk, D), lambda i: (i, 0)),
            pl.BlockSpec((blk, D), lambda i: (i, 0)),
            pl.BlockSpec((blk, 128), lambda i: (i, 0)),
            pl.BlockSpec((D, D), lambda i: (0, 0)),
            pl.BlockSpec((1, D), lambda i: (0, 0)),
            pl.BlockSpec((1, D), lambda i: (0, 0)),
            pl.BlockSpec((D, 4 * D), lambda i: (0, 0)),
            pl.BlockSpec((1, 4 * D), lambda i: (0, 0)),
            pl.BlockSpec((4 * D, D), lambda i: (0, 0)),
            pl.BlockSpec((1, D), lambda i: (0, 0)),
        ],
        out_specs=pl.BlockSpec((blk, D), lambda i: (i, 0)),
        out_shape=jax.ShapeDtypeStruct((S, D), F32),
    )(x, out_cmp, out_sel, out_win, gates, Wo, ln2_g, ln2_b, W1, b1, W2, b2)


@jax.jit
def _run(x, ln1_g, ln1_b, Wq, Wk, Wv, Wck, bck, Wcv, bcv, Wg, bg, Wo,
         ln2_g, ln2_b, W1, b1, W2, b2):
    x2d = x[0]
    Wg_pad = jnp.pad(Wg, ((0, 0), (0, 128 - 3 * H)))
    bcat = jnp.concatenate(
        [jnp.zeros((D + 2 * HKV * HD,), F32), bg,
         jnp.zeros((128 - 3 * H,), F32)])[None]
    Wcat = jnp.concatenate([Wq, Wk, Wv, Wg_pad], axis=1)

    Wck3 = Wck.reshape(L, HD, HD)
    Wcv3 = Wcv.reshape(L, HD, HD)
    z = jnp.zeros((L, HD, HD), F32)
    W_all = jnp.concatenate([
        jnp.concatenate([Wck3, z], axis=2),
        jnp.concatenate([z, Wcv3], axis=2),
    ], axis=1).reshape(L * 2 * HD, 2 * HD)
    bckv = jnp.concatenate([bck, bcv])[None]

    q, kvh, gates = _k1(x2d, ln1_g[None], ln1_b[None], Wcat, bcat)
    kvf = kvh.reshape(HKV, S // STRIDE, STRIDE * 2 * HD)

    out_cmp, top_idx = _kb(q, kvf, W_all, bckv)
    out_sel, out_win = _kc(top_idx.reshape(-1), q, kvh)
    out = _k7(x2d, out_cmp, out_sel, out_win, gates, Wo,
              ln2_g[None], ln2_b[None], W1, b1[None], W2, b2[None])
    return out[None]


def kernel(x, ln1_g, ln1_b, Wq, Wk, Wv, Wck, bck, Wcv, bcv, Wg, bg, Wo,
           ln2_g, ln2_b, W1, b1, W2, b2):
    return _run(x, ln1_g, ln1_b, Wq, Wk, Wv, Wck, bck, Wcv, bcv, Wg, bg, Wo,
                ln2_g, ln2_b, W1, b1, W2, b2)


# speedup vs baseline: 1.0406x; 1.0406x over previous
import jax
import jax.numpy as jnp
import numpy as np
from jax.experimental import pallas as pl
from jax.experimental.pallas import tpu as pltpu

D = 768
H = 12
HKV = 3
HPG = H // HKV
HD = 64
L = 32
STRIDE = 16
TOPN = 16
WIN = 512
S = 2048
NCMP = (S - L) // STRIDE + 1
NCMP_PAD = 128
NBLK = S // L
SCALE = 1.0 / np.sqrt(HD)

F32 = jnp.float32


def _ln(xb, g, b):
    m = jnp.mean(xb, axis=-1, keepdims=True)
    v = jnp.var(xb, axis=-1, keepdims=True)
    return (xb - m) * jax.lax.rsqrt(v + 1e-5) * g + b


def _k1_body(x_ref, g_ref, b_ref, w_ref, bc_ref, q_ref, kv_ref, gt_ref):
    xb = x_ref[:]
    ln = _ln(xb, g_ref[:], b_ref[:])
    out = jnp.dot(ln, w_ref[:], preferred_element_type=F32) + bc_ref[:]
    q_ref[:] = out[:, :D]
    for g in range(HKV):
        kv_ref[g] = jnp.concatenate(
            [out[:, D + g * HD:D + (g + 1) * HD],
             out[:, D + HKV * HD + g * HD:D + HKV * HD + (g + 1) * HD]],
            axis=1)
    gt_ref[:] = jax.nn.sigmoid(out[:, D + 2 * HKV * HD:])


def _k1(x, ln1_g, ln1_b, Wcat, bcat):
    blk = 256
    return pl.pallas_call(
        _k1_body,
        grid=(S // blk,),
        compiler_params=pltpu.CompilerParams(dimension_semantics=("parallel",)),
        in_specs=[
            pl.BlockSpec((blk, D), lambda i: (i, 0)),
            pl.BlockSpec((1, D), lambda i: (0, 0)),
            pl.BlockSpec((1, D), lambda i: (0, 0)),
            pl.BlockSpec(Wcat.shape, lambda i: (0, 0)),
            pl.BlockSpec((1, Wcat.shape[1]), lambda i: (0, 0)),
        ],
        out_specs=[
            pl.BlockSpec((blk, D), lambda i: (i, 0)),
            pl.BlockSpec((HKV, blk, 2 * HD), lambda i: (0, i, 0)),
            pl.BlockSpec((blk, 128), lambda i: (i, 0)),
        ],
        out_shape=[
            jax.ShapeDtypeStruct((S, D), F32),
            jax.ShapeDtypeStruct((HKV, S, 2 * HD), F32),
            jax.ShapeDtypeStruct((S, 128), F32),
        ],
    )(x, ln1_g, ln1_b, Wcat, bcat)


QC3 = 2048


def _kb_body(q_ref, kvf_ref, w_ref, b_ref, out_ref, idx_ref, ckv_s, impq_s):
    i = pl.program_id(1)
    nsteps = pl.num_programs(1)

    @pl.when(i == 0)
    def _():
        r = kvf_ref[0]
        t0 = jnp.dot(r, w_ref[:STRIDE * 2 * HD], preferred_element_type=F32)
        t1 = jnp.dot(r, w_ref[STRIDE * 2 * HD:], preferred_element_type=F32)
        zero = jnp.zeros((1, 2 * HD), F32)
        ckv_s[:] = t0 + jnp.concatenate([t1[1:], zero], axis=0) + b_ref[:]

    ckm = ckv_s[:, :HD]
    cvm = ckv_s[:, HD:]
    rows = jax.lax.broadcasted_iota(jnp.int32, (HPG * QC3, 1), 0)
    qpos = i * QC3 + rows % QC3
    nidx = jax.lax.broadcasted_iota(jnp.int32, (1, NCMP_PAD), 1)
    mask = qpos >= nidx * STRIDE + (L - 1)
    pad = nidx < NCMP

    q4 = jnp.concatenate(
        [q_ref[:, hp * HD:(hp + 1) * HD] for hp in range(HPG)], axis=0)
    s = jax.lax.dot_general(q4, ckm, (((1,), (1,)), ((), ())),
                            preferred_element_type=F32) * SCALE
    s = jnp.where(mask, s, -1e9)
    s = jnp.where(pad, s, -1e30)
    m = jnp.max(s, axis=-1, keepdims=True)
    p = jnp.exp(s - m)
    cp = p / jnp.sum(p, axis=-1, keepdims=True)
    o = jnp.dot(cp, cvm, preferred_element_type=F32)
    for hp in range(HPG):
        out_ref[:, hp * HD:(hp + 1) * HD] = o[hp * QC3:(hp + 1) * QC3, :]

    nq = QC3 // L
    ar = jax.lax.broadcasted_iota(jnp.int32, (nq, HPG * QC3), 0)
    ac = jax.lax.broadcasted_iota(jnp.int32, (nq, HPG * QC3), 1)
    A = jnp.where((ac % QC3) // L == ar, 1.0, 0.0).astype(F32)
    rr = jax.lax.broadcasted_iota(jnp.int32, (NCMP_PAD, NBLK), 0)
    cc = jax.lax.broadcasted_iota(jnp.int32, (NCMP_PAD, NBLK), 1)
    P = jnp.where((rr // 2 == cc) & (rr < NCMP), 1.0, 0.0).astype(F32)
    folded = jnp.dot(cp, P, preferred_element_type=F32)
    impq_s[pl.ds(i * nq, nq), :] = jnp.dot(A, folded,
                                           preferred_element_type=F32)

    @pl.when(i == nsteps - 1)
    def _():
        vals = impq_s[:]
        qb = jax.lax.broadcasted_iota(jnp.int32, (NBLK, NBLK), 0)
        mb = jax.lax.broadcasted_iota(jnp.int32, (NBLK, NBLK), 1)
        bonus = jnp.where((mb == qb) | (mb == 0), 1e6, 0.0).astype(F32)
        vals = jnp.where(qb >= mb, vals + bonus, -1e9)
        tcol = jax.lax.broadcasted_iota(jnp.int32, (NBLK, TOPN), 1)
        out = jnp.zeros((NBLK, TOPN), jnp.int32)
        for t in range(TOPN):
            m = jnp.argmax(vals, axis=1).astype(jnp.int32)
            out = jnp.where(tcol == t, m[:, None], out)
            vals = jnp.where(mb == m[:, None], -3e9, vals)
        idx_ref[0] = out


def _kb(q, kvf, W_all, bckv):
    return pl.pallas_call(
        _kb_body,
        grid=(HKV, S // QC3),
        compiler_params=pltpu.CompilerParams(
            dimension_semantics=("arbitrary", "arbitrary")),
        in_specs=[
            pl.BlockSpec((QC3, HPG * HD), lambda g, i: (i, g)),
            pl.BlockSpec((1, S // STRIDE, STRIDE * 2 * HD),
                         lambda g, i: (g, 0, 0)),
            pl.BlockSpec(W_all.shape, lambda g, i: (0, 0)),
            pl.BlockSpec((1, 2 * HD), lambda g, i: (0, 0)),
        ],
        out_specs=[
            pl.BlockSpec((QC3, HPG * HD), lambda g, i: (i, g)),
            pl.BlockSpec((1, NBLK, TOPN), lambda g, i: (g, 0, 0)),
        ],
        out_shape=[
            jax.ShapeDtypeStruct((S, D), F32),
            jax.ShapeDtypeStruct((HKV, NBLK, TOPN), jnp.int32),
        ],
        scratch_shapes=[
            pltpu.VMEM((NCMP_PAD, 2 * HD), F32),
            pltpu.VMEM((NBLK, NBLK), F32),
        ],
    )(q, kvf, W_all, bckv)


QC = 256
QB5 = QC // L
WK = WIN + QC


def _kc_body(idx_ref, q_ref, kv_ref, sel_ref, win_ref, kv_scr):
    g = pl.program_id(0)
    i = pl.program_id(1)

    rows = jax.lax.broadcasted_iota(jnp.int32, (HPG * L, 1), 0)
    jcol = jax.lax.broadcasted_iota(jnp.int32, (1, TOPN * L), 1)
    jmod = jcol % L

    for qq in range(QB5):
        qb = i * QB5 + qq
        base = g * NBLK * TOPN + qb * TOPN
        qpos = qb * L + rows % L

        colpos = jmod
        for t in range(TOPN):
            it = idx_ref[base + t]
            kv_scr[qq * TOPN * L + t * L:qq * TOPN * L + (t + 1) * L, :] = (
                kv_ref[0, pl.ds(it * L, L), :])
            colpos = colpos + jnp.where(jcol // L == t, it * L, 0)
        mask = colpos <= qpos

        ks = kv_scr[qq * TOPN * L:(qq + 1) * TOPN * L, :HD]
        vs = kv_scr[qq * TOPN * L:(qq + 1) * TOPN * L, HD:]
        q4 = jnp.concatenate(
            [q_ref[qq * L:(qq + 1) * L, hp * HD:(hp + 1) * HD]
             for hp in range(HPG)], axis=0)
        s = jax.lax.dot_general(q4, ks, (((1,), (1,)), ((), ())),
                                preferred_element_type=F32) * SCALE
        s = jnp.where(mask, s, -1e9)
        m = jnp.max(s, axis=-1, keepdims=True)
        p = jnp.exp(s - m)
        r = 1.0 / jnp.sum(p, axis=-1, keepdims=True)
        o = jnp.dot(p, vs, preferred_element_type=F32) * r
        for hp in range(HPG):
            sel_ref[qq * L:(qq + 1) * L, hp * HD:(hp + 1) * HD] = (
                o[hp * L:(hp + 1) * L, :])

    s0 = jnp.maximum(i - WIN // QC, 0) * QC
    kvw = kv_ref[0, pl.ds(s0, WK), :]
    kw = kvw[:, :HD]
    vw = kvw[:, HD:]
    wrows = jax.lax.broadcasted_iota(jnp.int32, (HPG * QC, 1), 0)
    qpos = i * QC + wrows % QC
    kpos = s0 + jax.lax.broadcasted_iota(jnp.int32, (1, WK), 1)
    wmask = (qpos >= kpos) & (qpos - kpos < WIN)
    q4 = jnp.concatenate(
        [q_ref[:, hp * HD:(hp + 1) * HD] for hp in range(HPG)], axis=0)
    s = jax.lax.dot_general(q4, kw, (((1,), (1,)), ((), ())),
                            preferred_element_type=F32) * SCALE
    s = jnp.where(wmask, s, -1e9)
    m = jnp.max(s, axis=-1, keepdims=True)
    p = jnp.exp(s - m)
    r = 1.0 / jnp.sum(p, axis=-1, keepdims=True)
    o = jnp.dot(p, vw, preferred_element_type=F32) * r
    for hp in range(HPG):
        win_ref[:, hp * HD:(hp + 1) * HD] = o[hp * QC:(hp + 1) * QC, :]


def _kc(top_idx_flat, q, kvh):
    grid_spec = pltpu.PrefetchScalarGridSpec(
        num_scalar_prefetch=1,
        grid=(HKV, S // QC),
        in_specs=[
            pl.BlockSpec((QC, HPG * HD), lambda g, i, *_: (i, g)),
            pl.BlockSpec((1, S, 2 * HD), lambda g, i, *_: (g, 0, 0)),
        ],
        out_specs=[
            pl.BlockSpec((QC, HPG * HD), lambda g, i, *_: (i, g)),
            pl.BlockSpec((QC, HPG * HD), lambda g, i, *_: (i, g)),
        ],
        scratch_shapes=[
            pltpu.VMEM((QB5 * TOPN * L, 2 * HD), F32),
        ],
    )
    return pl.pallas_call(
        _kc_body,
        grid_spec=grid_spec,
        compiler_params=pltpu.CompilerParams(
            dimension_semantics=("parallel", "parallel")),
        out_shape=[
            jax.ShapeDtypeStruct((S, D), F32),
            jax.ShapeDtypeStruct((S, D), F32),
        ],
    )(top_idx_flat, q, kvh)


def _k7_body(x_ref, cmp_ref, sel_ref, win_ref, g_ref, wo_ref,
             ln2g_ref, ln2b_ref, w1_ref, b1_ref, w2_ref, b2_ref, out_ref):
    gts = g_ref[:]
    rr = jax.lax.broadcasted_iota(jnp.int32, (128, D), 0)
    cc = jax.lax.broadcasted_iota(jnp.int32, (128, D), 1)
    head3 = 3 * (cc // HD)
    e0 = jnp.where(rr == head3, 1.0, 0.0).astype(F32)
    e1 = jnp.where(rr == head3 + 1, 1.0, 0.0).astype(F32)
    e2 = jnp.where(rr == head3 + 2, 1.0, 0.0).astype(F32)
    comb = (cmp_ref[:] * jnp.dot(gts, e0, preferred_element_type=F32)
            + sel_ref[:] * jnp.dot(gts, e1, preferred_element_type=F32)
            + win_ref[:] * jnp.dot(gts, e2, preferred_element_type=F32))
    x1 = x_ref[:] + jnp.dot(comb, wo_ref[:], preferred_element_type=F32)
    ln = _ln(x1, ln2g_ref[:], ln2b_ref[:])
    h = jax.nn.gelu(jnp.dot(ln, w1_ref[:], preferred_element_type=F32) + b1_ref[:])
    out_ref[:] = x1 + jnp.dot(h, w2_ref[:], preferred_element_type=F32) + b2_ref[:]


def _k7(x, out_cmp, out_sel, out_win, gates, Wo, ln2_g, ln2_b, W1, b1, W2, b2):
    blk = 256
    return pl.pallas_call(
        _k7_body,
        grid=(S // blk,),
        compiler_params=pltpu.CompilerParams(dimension_semantics=("parallel",)),
        in_specs=[
            pl.BlockSpec((blk, D), lambda i: (i, 0)),
            pl.BlockSpec((blk, D), lambda i: (i, 0)),
            pl.BlockSpec((blk, D), lambda i: (i, 0)),
            pl.BlockSpec((blk, D), lambda i: (i, 0)),
            pl.BlockSpec((blk, 128), lambda i: (i, 0)),
            pl.BlockSpec((D, D), lambda i: (0, 0)),
            pl.BlockSpec((1, D), lambda i: (0, 0)),
            pl.BlockSpec((1, D), lambda i: (0, 0)),
            pl.BlockSpec((D, 4 * D), lambda i: (0, 0)),
            pl.BlockSpec((1, 4 * D), lambda i: (0, 0)),
            pl.BlockSpec((4 * D, D), lambda i: (0, 0)),
            pl.BlockSpec((1, D), lambda i: (0, 0)),
        ],
        out_specs=pl.BlockSpec((blk, D), lambda i: (i, 0)),
        out_shape=jax.ShapeDtypeStruct((S, D), F32),
    )(x, out_cmp, out_sel, out_win, gates, Wo, ln2_g, ln2_b, W1, b1, W2, b2)


@jax.jit
def _run(x, ln1_g, ln1_b, Wq, Wk, Wv, Wck, bck, Wcv, bcv, Wg, bg, Wo,
         ln2_g, ln2_b, W1, b1, W2, b2):
    x2d = x[0]
    Wg_pad = jnp.pad(Wg, ((0, 0), (0, 128 - 3 * H)))
    bcat = jnp.concatenate(
        [jnp.zeros((D + 2 * HKV * HD,), F32), bg,
         jnp.zeros((128 - 3 * H,), F32)])[None]
    Wcat = jnp.concatenate([Wq, Wk, Wv, Wg_pad], axis=1)

    Wck3 = Wck.reshape(L, HD, HD)
    Wcv3 = Wcv.reshape(L, HD, HD)
    z = jnp.zeros((L, HD, HD), F32)
    W_all = jnp.concatenate([
        jnp.concatenate([Wck3, z], axis=2),
        jnp.concatenate([z, Wcv3], axis=2),
    ], axis=1).reshape(L * 2 * HD, 2 * HD)
    bckv = jnp.concatenate([bck, bcv])[None]

    q, kvh, gates = _k1(x2d, ln1_g[None], ln1_b[None], Wcat, bcat)
    kvf = kvh.reshape(HKV, S // STRIDE, STRIDE * 2 * HD)

    out_cmp, top_idx = _kb(q, kvf, W_all, bckv)
    out_sel, out_win = _kc(top_idx.reshape(-1), q, kvh)
    out = _k7(x2d, out_cmp, out_sel, out_win, gates, Wo,
              ln2_g[None], ln2_b[None], W1, b1[None], W2, b2[None])
    return out[None]


def kernel(x, ln1_g, ln1_b, Wq, Wk, Wv, Wck, bck, Wcv, bcv, Wg, bg, Wo,
           ln2_g, ln2_b, W1, b1, W2, b2):
    return _run(x, ln1_g, ln1_b, Wq, Wk, Wv, Wck, bck, Wcv, bcv, Wg, bg, Wo,
                ln2_g, ln2_b, W1, b1, W2, b2)


# speedup vs baseline: 1.0463x; 1.0054x over previous
import jax
import jax.numpy as jnp
import numpy as np
from jax.experimental import pallas as pl
from jax.experimental.pallas import tpu as pltpu

D = 768
H = 12
HKV = 3
HPG = H // HKV
HD = 64
L = 32
STRIDE = 16
TOPN = 16
WIN = 512
S = 2048
NCMP = (S - L) // STRIDE + 1
NCMP_PAD = 128
NBLK = S // L
SCALE = 1.0 / np.sqrt(HD)

F32 = jnp.float32


def _ln(xb, g, b):
    m = jnp.mean(xb, axis=-1, keepdims=True)
    v = jnp.var(xb, axis=-1, keepdims=True)
    return (xb - m) * jax.lax.rsqrt(v + 1e-5) * g + b


def _k1_body(x_ref, g_ref, b_ref, w_ref, bc_ref, q_ref, kv_ref, gt_ref):
    xb = x_ref[:]
    ln = _ln(xb, g_ref[:], b_ref[:])
    out = jnp.dot(ln, w_ref[:], preferred_element_type=F32) + bc_ref[:]
    q_ref[:] = out[:, :D]
    for g in range(HKV):
        kv_ref[g] = jnp.concatenate(
            [out[:, D + g * HD:D + (g + 1) * HD],
             out[:, D + HKV * HD + g * HD:D + HKV * HD + (g + 1) * HD]],
            axis=1)
    gt_ref[:] = jax.nn.sigmoid(out[:, D + 2 * HKV * HD:])


def _k1(x, ln1_g, ln1_b, Wcat, bcat):
    blk = 256
    return pl.pallas_call(
        _k1_body,
        grid=(S // blk,),
        compiler_params=pltpu.CompilerParams(dimension_semantics=("parallel",)),
        in_specs=[
            pl.BlockSpec((blk, D), lambda i: (i, 0)),
            pl.BlockSpec((1, D), lambda i: (0, 0)),
            pl.BlockSpec((1, D), lambda i: (0, 0)),
            pl.BlockSpec(Wcat.shape, lambda i: (0, 0)),
            pl.BlockSpec((1, Wcat.shape[1]), lambda i: (0, 0)),
        ],
        out_specs=[
            pl.BlockSpec((blk, D), lambda i: (i, 0)),
            pl.BlockSpec((HKV, blk, 2 * HD), lambda i: (0, i, 0)),
            pl.BlockSpec((blk, 128), lambda i: (i, 0)),
        ],
        out_shape=[
            jax.ShapeDtypeStruct((S, D), F32),
            jax.ShapeDtypeStruct((HKV, S, 2 * HD), F32),
            jax.ShapeDtypeStruct((S, 128), F32),
        ],
    )(x, ln1_g, ln1_b, Wcat, bcat)


QC3 = 2048


def _kb_body(q_ref, kvf_ref, w_ref, b_ref, out_ref, idx_ref, ckv_s, impq_s):
    i = pl.program_id(1)
    nsteps = pl.num_programs(1)

    @pl.when(i == 0)
    def _():
        r = kvf_ref[0]
        t0 = jnp.dot(r, w_ref[:STRIDE * 2 * HD], preferred_element_type=F32)
        t1 = jnp.dot(r, w_ref[STRIDE * 2 * HD:], preferred_element_type=F32)
        zero = jnp.zeros((1, 2 * HD), F32)
        ckv_s[:] = t0 + jnp.concatenate([t1[1:], zero], axis=0) + b_ref[:]

    ckm = ckv_s[:, :HD]
    cvm = ckv_s[:, HD:]
    rows = jax.lax.broadcasted_iota(jnp.int32, (HPG * QC3, 1), 0)
    qpos = i * QC3 + rows % QC3
    nidx = jax.lax.broadcasted_iota(jnp.int32, (1, NCMP_PAD), 1)
    mask = qpos >= nidx * STRIDE + (L - 1)
    pad = nidx < NCMP

    q4 = jnp.concatenate(
        [q_ref[:, hp * HD:(hp + 1) * HD] for hp in range(HPG)], axis=0)
    s = jax.lax.dot_general(q4, ckm, (((1,), (1,)), ((), ())),
                            preferred_element_type=F32) * SCALE
    s = jnp.where(mask, s, -1e9)
    s = jnp.where(pad, s, -1e30)
    m = jnp.max(s, axis=-1, keepdims=True)
    p = jnp.exp(s - m)
    cp = p / jnp.sum(p, axis=-1, keepdims=True)
    o = jnp.dot(cp, cvm, preferred_element_type=F32)
    for hp in range(HPG):
        out_ref[:, hp * HD:(hp + 1) * HD] = o[hp * QC3:(hp + 1) * QC3, :]

    nq = QC3 // L
    ar = jax.lax.broadcasted_iota(jnp.int32, (nq, HPG * QC3), 0)
    ac = jax.lax.broadcasted_iota(jnp.int32, (nq, HPG * QC3), 1)
    A = jnp.where((ac % QC3) // L == ar, 1.0, 0.0).astype(F32)
    rr = jax.lax.broadcasted_iota(jnp.int32, (NCMP_PAD, NBLK), 0)
    cc = jax.lax.broadcasted_iota(jnp.int32, (NCMP_PAD, NBLK), 1)
    P = jnp.where((rr // 2 == cc) & (rr < NCMP), 1.0, 0.0).astype(F32)
    folded = jnp.dot(cp, P, preferred_element_type=F32)
    impq_s[pl.ds(i * nq, nq), :] = jnp.dot(A, folded,
                                           preferred_element_type=F32)

    @pl.when(i == nsteps - 1)
    def _():
        vals = impq_s[:]
        qb = jax.lax.broadcasted_iota(jnp.int32, (NBLK, NBLK), 0)
        mb = jax.lax.broadcasted_iota(jnp.int32, (NBLK, NBLK), 1)
        bonus = jnp.where((mb == qb) | (mb == 0), 1e6, 0.0).astype(F32)
        vals = jnp.where(qb >= mb, vals + bonus, -1e9)
        tcol = jax.lax.broadcasted_iota(jnp.int32, (NBLK, TOPN), 1)
        out = jnp.zeros((NBLK, TOPN), jnp.int32)
        for t in range(TOPN):
            m = jnp.argmax(vals, axis=1).astype(jnp.int32)
            out = jnp.where(tcol == t, m[:, None], out)
            vals = jnp.where(mb == m[:, None], -3e9, vals)
        idx_ref[0] = out


def _kb(q, kvf, W_all, bckv):
    return pl.pallas_call(
        _kb_body,
        grid=(HKV, S // QC3),
        compiler_params=pltpu.CompilerParams(
            dimension_semantics=("arbitrary", "arbitrary")),
        in_specs=[
            pl.BlockSpec((QC3, HPG * HD), lambda g, i: (i, g)),
            pl.BlockSpec((1, S // STRIDE, STRIDE * 2 * HD),
                         lambda g, i: (g, 0, 0)),
            pl.BlockSpec(W_all.shape, lambda g, i: (0, 0)),
            pl.BlockSpec((1, 2 * HD), lambda g, i: (0, 0)),
        ],
        out_specs=[
            pl.BlockSpec((QC3, HPG * HD), lambda g, i: (i, g)),
            pl.BlockSpec((1, NBLK, TOPN), lambda g, i: (g, 0, 0)),
        ],
        out_shape=[
            jax.ShapeDtypeStruct((S, D), F32),
            jax.ShapeDtypeStruct((HKV, NBLK, TOPN), jnp.int32),
        ],
        scratch_shapes=[
            pltpu.VMEM((NCMP_PAD, 2 * HD), F32),
            pltpu.VMEM((NBLK, NBLK), F32),
        ],
    )(q, kvf, W_all, bckv)


QC = 256
QB5 = QC // L
WK = WIN + QC


def _kc_body(idx_ref, q_ref, kv_ref, sel_ref, win_ref, kv_scr):
    g = pl.program_id(0)
    i = pl.program_id(1)

    rows1 = jax.lax.broadcasted_iota(jnp.int32, (L, 1), 0)
    jcol = jax.lax.broadcasted_iota(jnp.int32, (1, TOPN * L), 1)
    jmod = jcol % L

    for qq in range(QB5):
        qb = i * QB5 + qq
        base = g * NBLK * TOPN + qb * TOPN
        qpos = qb * L + rows1

        colpos = jmod
        for t in range(TOPN):
            it = idx_ref[base + t]
            kv_scr[qq * TOPN * L + t * L:qq * TOPN * L + (t + 1) * L, :] = (
                kv_ref[0, pl.ds(it * L, L), :])
            colpos = colpos + jnp.where(jcol // L == t, it * L, 0)
        mask = colpos <= qpos

        ks = kv_scr[qq * TOPN * L:(qq + 1) * TOPN * L, :HD]
        vs = kv_scr[qq * TOPN * L:(qq + 1) * TOPN * L, HD:]
        q4 = jnp.concatenate(
            [q_ref[qq * L:(qq + 1) * L, hp * HD:(hp + 1) * HD]
             for hp in range(HPG)], axis=0)
        s = jax.lax.dot_general(q4, ks, (((1,), (1,)), ((), ())),
                                preferred_element_type=F32) * SCALE
        s = jnp.where(mask[None], s.reshape(HPG, L, TOPN * L),
                      -1e9).reshape(HPG * L, TOPN * L)
        m = jnp.max(s, axis=-1, keepdims=True)
        p = jnp.exp(s - m)
        r = 1.0 / jnp.sum(p, axis=-1, keepdims=True)
        o = jnp.dot(p, vs, preferred_element_type=F32) * r
        for hp in range(HPG):
            sel_ref[qq * L:(qq + 1) * L, hp * HD:(hp + 1) * HD] = (
                o[hp * L:(hp + 1) * L, :])

    s0 = jnp.maximum(i - WIN // QC, 0) * QC
    kvw = kv_ref[0, pl.ds(s0, WK), :]
    kw = kvw[:, :HD]
    vw = kvw[:, HD:]
    qpos = i * QC + jax.lax.broadcasted_iota(jnp.int32, (QC, 1), 0)
    kpos = s0 + jax.lax.broadcasted_iota(jnp.int32, (1, WK), 1)
    diff = qpos - kpos
    wmask = (diff >= 0) & (diff < WIN)
    q4 = jnp.concatenate(
        [q_ref[:, hp * HD:(hp + 1) * HD] for hp in range(HPG)], axis=0)
    s = jax.lax.dot_general(q4, kw, (((1,), (1,)), ((), ())),
                            preferred_element_type=F32) * SCALE
    s = jnp.where(wmask[None], s.reshape(HPG, QC, WK),
                  -1e9).reshape(HPG * QC, WK)
    m = jnp.max(s, axis=-1, keepdims=True)
    p = jnp.exp(s - m)
    r = 1.0 / jnp.sum(p, axis=-1, keepdims=True)
    o = jnp.dot(p, vw, preferred_element_type=F32) * r
    for hp in range(HPG):
        win_ref[:, hp * HD:(hp + 1) * HD] = o[hp * QC:(hp + 1) * QC, :]


def _kc(top_idx_flat, q, kvh):
    grid_spec = pltpu.PrefetchScalarGridSpec(
        num_scalar_prefetch=1,
        grid=(HKV, S // QC),
        in_specs=[
            pl.BlockSpec((QC, HPG * HD), lambda g, i, *_: (i, g)),
            pl.BlockSpec((1, S, 2 * HD), lambda g, i, *_: (g, 0, 0)),
        ],
        out_specs=[
            pl.BlockSpec((QC, HPG * HD), lambda g, i, *_: (i, g)),
            pl.BlockSpec((QC, HPG * HD), lambda g, i, *_: (i, g)),
        ],
        scratch_shapes=[
            pltpu.VMEM((QB5 * TOPN * L, 2 * HD), F32),
        ],
    )
    return pl.pallas_call(
        _kc_body,
        grid_spec=grid_spec,
        compiler_params=pltpu.CompilerParams(
            dimension_semantics=("parallel", "parallel")),
        out_shape=[
            jax.ShapeDtypeStruct((S, D), F32),
            jax.ShapeDtypeStruct((S, D), F32),
        ],
    )(top_idx_flat, q, kvh)


def _k7_body(x_ref, cmp_ref, sel_ref, win_ref, g_ref, wo_ref,
             ln2g_ref, ln2b_ref, w1_ref, b1_ref, w2_ref, b2_ref, out_ref):
    gts = g_ref[:]
    rr = jax.lax.broadcasted_iota(jnp.int32, (128, D), 0)
    cc = jax.lax.broadcasted_iota(jnp.int32, (128, D), 1)
    head3 = 3 * (cc // HD)
    e0 = jnp.where(rr == head3, 1.0, 0.0).astype(F32)
    e1 = jnp.where(rr == head3 + 1, 1.0, 0.0).astype(F32)
    e2 = jnp.where(rr == head3 + 2, 1.0, 0.0).astype(F32)
    comb = (cmp_ref[:] * jnp.dot(gts, e0, preferred_element_type=F32)
            + sel_ref[:] * jnp.dot(gts, e1, preferred_element_type=F32)
            + win_ref[:] * jnp.dot(gts, e2, preferred_element_type=F32))
    x1 = x_ref[:] + jnp.dot(comb, wo_ref[:], preferred_element_type=F32)
    ln = _ln(x1, ln2g_ref[:], ln2b_ref[:])
    h = jax.nn.gelu(jnp.dot(ln, w1_ref[:], preferred_element_type=F32) + b1_ref[:])
    out_ref[:] = x1 + jnp.dot(h, w2_ref[:], preferred_element_type=F32) + b2_ref[:]


def _k7(x, out_cmp, out_sel, out_win, gates, Wo, ln2_g, ln2_b, W1, b1, W2, b2):
    blk = 256
    return pl.pallas_call(
        _k7_body,
        grid=(S // blk,),
        compiler_params=pltpu.CompilerParams(dimension_semantics=("parallel",)),
        in_specs=[
            pl.BlockSpec((blk, D), lambda i: (i, 0)),
            pl.BlockSpec((blk, D), lambda i: (i, 0)),
            pl.BlockSpec((blk, D), lambda i: (i, 0)),
            pl.BlockSpec((blk, D), lambda i: (i, 0)),
            pl.BlockSpec((blk, 128), lambda i: (i, 0)),
            pl.BlockSpec((D, D), lambda i: (0, 0)),
            pl.BlockSpec((1, D), lambda i: (0, 0)),
            pl.BlockSpec((1, D), lambda i: (0, 0)),
            pl.BlockSpec((D, 4 * D), lambda i: (0, 0)),
            pl.BlockSpec((1, 4 * D), lambda i: (0, 0)),
            pl.BlockSpec((4 * D, D), lambda i: (0, 0)),
            pl.BlockSpec((1, D), lambda i: (0, 0)),
        ],
        out_specs=pl.BlockSpec((blk, D), lambda i: (i, 0)),
        out_shape=jax.ShapeDtypeStruct((S, D), F32),
    )(x, out_cmp, out_sel, out_win, gates, Wo, ln2_g, ln2_b, W1, b1, W2, b2)


@jax.jit
def _run(x, ln1_g, ln1_b, Wq, Wk, Wv, Wck, bck, Wcv, bcv, Wg, bg, Wo,
         ln2_g, ln2_b, W1, b1, W2, b2):
    x2d = x[0]
    Wg_pad = jnp.pad(Wg, ((0, 0), (0, 128 - 3 * H)))
    bcat = jnp.concatenate(
        [jnp.zeros((D + 2 * HKV * HD,), F32), bg,
         jnp.zeros((128 - 3 * H,), F32)])[None]
    Wcat = jnp.concatenate([Wq, Wk, Wv, Wg_pad], axis=1)

    Wck3 = Wck.reshape(L, HD, HD)
    Wcv3 = Wcv.reshape(L, HD, HD)
    z = jnp.zeros((L, HD, HD), F32)
    W_all = jnp.concatenate([
        jnp.concatenate([Wck3, z], axis=2),
        jnp.concatenate([z, Wcv3], axis=2),
    ], axis=1).reshape(L * 2 * HD, 2 * HD)
    bckv = jnp.concatenate([bck, bcv])[None]

    q, kvh, gates = _k1(x2d, ln1_g[None], ln1_b[None], Wcat, bcat)
    kvf = kvh.reshape(HKV, S // STRIDE, STRIDE * 2 * HD)

    out_cmp, top_idx = _kb(q, kvf, W_all, bckv)
    out_sel, out_win = _kc(top_idx.reshape(-1), q, kvh)
    out = _k7(x2d, out_cmp, out_sel, out_win, gates, Wo,
              ln2_g[None], ln2_b[None], W1, b1[None], W2, b2[None])
    return out[None]


def kernel(x, ln1_g, ln1_b, Wq, Wk, Wv, Wck, bck, Wcv, bcv, Wg, bg, Wo,
           ln2_g, ln2_b, W1, b1, W2, b2):
    return _run(x, ln1_g, ln1_b, Wq, Wk, Wv, Wck, bck, Wcv, bcv, Wg, bg, Wo,
                ln2_g, ln2_b, W1, b1, W2, b2)
